# Initial kernel scaffold; baseline (speedup 1.0000x reference)
import jax
import jax.numpy as jnp
from jax.experimental import pallas as pl


def kernel(reads, edge_index, overlap_similarity, overlap_length, emb, conv_w, conv_b, edge_w, edge_b, A_w, A_b, B_w, B_b, C_w, C_b, Dm_w, Dm_b, Em_w, Em_b, dec_w1, dec_b1, dec_w2, dec_b2):
    raise NotImplementedError("write your pallas kernel here")


# speedup vs baseline: 1.0603x; 1.0603x over previous
import functools
import jax
import jax.numpy as jnp
from jax.experimental import pallas as pl
from jax.experimental.pallas import tpu as pltpu

N = 10000
D = 128
L = 128
K = 20
T = L - K + 1
TP = 112


def _row_call(fn, nrows, out_cols, blocked, full, bm):
    grid = (pl.cdiv(nrows, bm),)
    in_specs = [pl.BlockSpec((bm, a.shape[1]), lambda i: (i, 0)) for a in blocked]
    in_specs += [pl.BlockSpec(f.shape, lambda i, _nd=f.ndim: (0,) * _nd) for f in full]
    out_specs = [pl.BlockSpec((bm, c), lambda i: (i, 0)) for c in out_cols]
    out_shape = [jax.ShapeDtypeStruct((nrows, c), jnp.float32) for c in out_cols]

    def body(*refs):
        ins = refs[: len(blocked) + len(full)]
        outs = refs[len(blocked) + len(full):]
        vals = [r[...] for r in ins]
        res = fn(*vals)
        if not isinstance(res, (tuple, list)):
            res = (res,)
        for o, r in zip(outs, res):
            o[...] = r

    res = pl.pallas_call(body, grid=grid, in_specs=in_specs,
                         out_specs=out_specs, out_shape=out_shape)(*blocked, *full)
    return res


def _encoder_body(ru_ref, emb_ref, w_ref, b_ref, out_ref):
    r = ru_ref[...]
    parts = []
    for c in range(3):
        xc = jnp.zeros(r.shape, jnp.float32)
        for v in range(5):
            xc = xc + jnp.where(r == v, emb_ref[v, c], 0.0)
        parts.append(xc)
    x = jnp.concatenate(parts, axis=-1)
    bn = x.shape[0]
    xr = x.reshape(bn * TP, 3 * K)
    y = jnp.dot(xr, w_ref[...], preferred_element_type=jnp.float32)
    y = jnp.maximum(y + b_ref[...], 0.0)
    out_ref[...] = jnp.max(y.reshape(bn, TP, D), axis=1)


def _encode(reads, emb, conv_w, conv_b):
    ru = jnp.stack([reads[:, k:k + T] for k in range(K)], axis=-1)
    ru = jnp.concatenate([ru, ru[:, :TP - T, :]], axis=1)
    wp = conv_w.transpose(1, 2, 0).reshape(3 * K, D)
    bn = 32
    grid = (pl.cdiv(N, bn),)
    h = pl.pallas_call(
        _encoder_body, grid=grid,
        in_specs=[
            pl.BlockSpec((bn, TP, K), lambda i: (i, 0, 0)),
            pl.BlockSpec(memory_space=pltpu.SMEM),
            pl.BlockSpec((3 * K, D), lambda i: (0, 0)),
            pl.BlockSpec((1, D), lambda i: (0, 0)),
        ],
        out_specs=pl.BlockSpec((bn, D), lambda i: (i, 0)),
        out_shape=jax.ShapeDtypeStruct((N, D), jnp.float32),
    )(ru, emb, wp, conv_b.reshape(1, D))
    return h


def _sigmoid(x):
    return 1.0 / (1.0 + jnp.exp(-x))


def kernel(reads, edge_index, overlap_similarity, overlap_length, emb, conv_w,
           conv_b, edge_w, edge_b, A_w, A_b, B_w, B_b, C_w, C_b, Dm_w, Dm_b,
           Em_w, Em_b, dec_w1, dec_b1, dec_w2, dec_b2):
    E = overlap_similarity.shape[0]
    reads = reads.astype(jnp.int32)
    src = edge_index[0]
    dst = edge_index[1]

    h = _encode(reads, emb, conv_w, conv_b)

    ov = jnp.stack([overlap_similarity, overlap_length], axis=1)

    def edge_enc(o, ew, eb):
        return o[:, 0:1] * ew[0:1, :] + o[:, 1:2] * ew[1:2, :] + eb

    (e,) = _row_call(edge_enc, E, [D], [ov], [edge_w, edge_b.reshape(1, D)], 2048)

    def node_mm(hv, w4, b4):
        return jnp.dot(hv, w4, preferred_element_type=jnp.float32) + b4

    def edge_stage(ev, dhs, ehd, bhs, cw, cb):
        ce = jnp.dot(ev, cw, preferred_element_type=jnp.float32) + cb
        en = ce + dhs + ehd
        sig = _sigmoid(en)
        return sig, sig * bhs, ev + jnp.maximum(en, 0.0)

    def h_update(hv, ah, num, den):
        return hv + jnp.maximum(ah + num / (den + 1e-6), 0.0)

    for l in range(4):
        w4 = jnp.concatenate([A_w[l], B_w[l], Dm_w[l], Em_w[l]], axis=1)
        b4 = jnp.concatenate([A_b[l], B_b[l], Dm_b[l], Em_b[l]]).reshape(1, 4 * D)
        (hw,) = _row_call(node_mm, N, [4 * D], [h], [w4, b4], 1024)
        Ah, Bh, Dh, Eh = (hw[:, 0:D], hw[:, D:2 * D], hw[:, 2 * D:3 * D],
                          hw[:, 3 * D:4 * D])
        dhs = jnp.take(Dh, src, axis=0)
        ehd = jnp.take(Eh, dst, axis=0)
        bhs = jnp.take(Bh, src, axis=0)
        sig, sigb, e = _row_call(edge_stage, E, [D, D, D], [e, dhs, ehd, bhs],
                                 [C_w[l], C_b[l].reshape(1, D)], 1024)
        num = jax.ops.segment_sum(sigb, dst, num_segments=N)
        den = jax.ops.segment_sum(sig, dst, num_segments=N)
        (h,) = _row_call(h_update, N, [D], [h, Ah, num, den], [], 1024)

    hs = jnp.take(h, src, axis=0)
    hd = jnp.take(h, dst, axis=0)

    def dec(a, b, c, w1a, w1b, w1c, b1, w2r, b2):
        z = (jnp.dot(a, w1a, preferred_element_type=jnp.float32)
             + jnp.dot(b, w1b, preferred_element_type=jnp.float32)
             + jnp.dot(c, w1c, preferred_element_type=jnp.float32) + b1)
        z = jnp.maximum(z, 0.0)
        return jnp.sum(z * w2r, axis=1, keepdims=True) + b2

    (p,) = _row_call(dec, E, [1], [hs, hd, e],
                     [dec_w1[0:D], dec_w1[D:2 * D], dec_w1[2 * D:3 * D],
                      dec_b1.reshape(1, D), dec_w2.reshape(1, D),
                      dec_b2.reshape(1, 1)], 1024)
    return p[:, 0]


# speedup vs baseline: 1.0756x; 1.0144x over previous
import functools
import jax
import jax.numpy as jnp
from jax.experimental import pallas as pl
from jax.experimental.pallas import tpu as pltpu

N = 10000
D = 128
L = 128
K = 20
T = L - K + 1
TP = 112


def _row_call(fn, nrows, out_cols, blocked, full, bm):
    grid = (pl.cdiv(nrows, bm),)
    in_specs = [pl.BlockSpec((bm, a.shape[1]), lambda i: (i, 0)) for a in blocked]
    in_specs += [pl.BlockSpec(f.shape, lambda i, _nd=f.ndim: (0,) * _nd) for f in full]
    out_specs = [pl.BlockSpec((bm, c), lambda i: (i, 0)) for c in out_cols]
    out_shape = [jax.ShapeDtypeStruct((nrows, c), jnp.float32) for c in out_cols]

    def body(*refs):
        ins = refs[: len(blocked) + len(full)]
        outs = refs[len(blocked) + len(full):]
        vals = [r[...] for r in ins]
        res = fn(*vals)
        if not isinstance(res, (tuple, list)):
            res = (res,)
        for o, r in zip(outs, res):
            o[...] = r

    res = pl.pallas_call(body, grid=grid, in_specs=in_specs,
                         out_specs=out_specs, out_shape=out_shape)(*blocked, *full)
    return res


def _encoder_body(ru_ref, emb_ref, w_ref, b_ref, out_ref):
    r = ru_ref[...]
    parts = []
    for c in range(3):
        xc = jnp.zeros(r.shape, jnp.float32)
        for v in range(5):
            xc = xc + jnp.where(r == v, emb_ref[v, c], 0.0)
        parts.append(xc)
    x = jnp.concatenate(parts, axis=-1)
    bn = x.shape[0]
    xr = x.reshape(bn * TP, 3 * K)
    y = jnp.dot(xr, w_ref[...], preferred_element_type=jnp.float32)
    y = jnp.maximum(y + b_ref[...], 0.0)
    out_ref[...] = jnp.max(y.reshape(bn, TP, D), axis=1)


def _encode(reads, emb, conv_w, conv_b):
    ru = jnp.stack([reads[:, k:k + T] for k in range(K)], axis=-1)
    ru = jnp.concatenate([ru, ru[:, :TP - T, :]], axis=1)
    wp = conv_w.transpose(1, 2, 0).reshape(3 * K, D)
    bn = 80
    grid = (pl.cdiv(N, bn),)
    h = pl.pallas_call(
        _encoder_body, grid=grid,
        in_specs=[
            pl.BlockSpec((bn, TP, K), lambda i: (i, 0, 0)),
            pl.BlockSpec(memory_space=pltpu.SMEM),
            pl.BlockSpec((3 * K, D), lambda i: (0, 0)),
            pl.BlockSpec((1, D), lambda i: (0, 0)),
        ],
        out_specs=pl.BlockSpec((bn, D), lambda i: (i, 0)),
        out_shape=jax.ShapeDtypeStruct((N, D), jnp.float32),
    )(ru, emb, wp, conv_b.reshape(1, D))
    return h


def _sigmoid(x):
    return 1.0 / (1.0 + jnp.exp(-x))


def kernel(reads, edge_index, overlap_similarity, overlap_length, emb, conv_w,
           conv_b, edge_w, edge_b, A_w, A_b, B_w, B_b, C_w, C_b, Dm_w, Dm_b,
           Em_w, Em_b, dec_w1, dec_b1, dec_w2, dec_b2):
    E = overlap_similarity.shape[0]
    reads = reads.astype(jnp.int32)
    src = edge_index[0]
    dst = edge_index[1]

    h = _encode(reads, emb, conv_w, conv_b)

    ov = jnp.stack([overlap_similarity, overlap_length], axis=1)

    def edge_enc(o, ew, eb):
        return o[:, 0:1] * ew[0:1, :] + o[:, 1:2] * ew[1:2, :] + eb

    (e,) = _row_call(edge_enc, E, [D], [ov], [edge_w, edge_b.reshape(1, D)], 2048)

    def node_mm(hv, w4, b4):
        return jnp.dot(hv, w4, preferred_element_type=jnp.float32) + b4

    def edge_stage(ev, dhs, ehd, bhs, cw, cb):
        ce = jnp.dot(ev, cw, preferred_element_type=jnp.float32) + cb
        en = ce + dhs + ehd
        sig = _sigmoid(en)
        return sig, sig * bhs, ev + jnp.maximum(en, 0.0)

    def h_update(hv, ah, num, den):
        return hv + jnp.maximum(ah + num / (den + 1e-6), 0.0)

    for l in range(4):
        w4 = jnp.concatenate([A_w[l], B_w[l], Dm_w[l], Em_w[l]], axis=1)
        b4 = jnp.concatenate([A_b[l], B_b[l], Dm_b[l], Em_b[l]]).reshape(1, 4 * D)
        (hw,) = _row_call(node_mm, N, [4 * D], [h], [w4, b4], 1024)
        Ah, Bh, Dh, Eh = (hw[:, 0:D], hw[:, D:2 * D], hw[:, 2 * D:3 * D],
                          hw[:, 3 * D:4 * D])
        dhs = jnp.take(Dh, src, axis=0)
        ehd = jnp.take(Eh, dst, axis=0)
        bhs = jnp.take(Bh, src, axis=0)
        sig, sigb, e = _row_call(edge_stage, E, [D, D, D], [e, dhs, ehd, bhs],
                                 [C_w[l], C_b[l].reshape(1, D)], 2048)
        num = jax.ops.segment_sum(sigb, dst, num_segments=N)
        den = jax.ops.segment_sum(sig, dst, num_segments=N)
        (h,) = _row_call(h_update, N, [D], [h, Ah, num, den], [], 1024)

    hs = jnp.take(h, src, axis=0)
    hd = jnp.take(h, dst, axis=0)

    def dec(a, b, c, w1a, w1b, w1c, b1, w2r, b2):
        z = (jnp.dot(a, w1a, preferred_element_type=jnp.float32)
             + jnp.dot(b, w1b, preferred_element_type=jnp.float32)
             + jnp.dot(c, w1c, preferred_element_type=jnp.float32) + b1)
        z = jnp.maximum(z, 0.0)
        return jnp.sum(z * w2r, axis=1, keepdims=True) + b2

    (p,) = _row_call(dec, E, [1], [hs, hd, e],
                     [dec_w1[0:D], dec_w1[D:2 * D], dec_w1[2 * D:3 * D],
                      dec_b1.reshape(1, D), dec_w2.reshape(1, D),
                      dec_b2.reshape(1, 1)], 1024)
    return p[:, 0]


# speedup vs baseline: 1.1696x; 1.0874x over previous
import functools
import jax
import jax.numpy as jnp
from jax.experimental import pallas as pl
from jax.experimental.pallas import tpu as pltpu

N = 10000
D = 128
L = 128
K = 20
T = L - K + 1
TP = 112


def _row_call(fn, nrows, out_cols, blocked, full, bm):
    grid = (pl.cdiv(nrows, bm),)
    in_specs = [pl.BlockSpec((bm, a.shape[1]), lambda i: (i, 0)) for a in blocked]
    in_specs += [pl.BlockSpec(f.shape, lambda i, _nd=f.ndim: (0,) * _nd) for f in full]
    out_specs = [pl.BlockSpec((bm, c), lambda i: (i, 0)) for c in out_cols]
    out_shape = [jax.ShapeDtypeStruct((nrows, c), jnp.float32) for c in out_cols]

    def body(*refs):
        ins = refs[: len(blocked) + len(full)]
        outs = refs[len(blocked) + len(full):]
        vals = [r[...] for r in ins]
        res = fn(*vals)
        if not isinstance(res, (tuple, list)):
            res = (res,)
        for o, r in zip(outs, res):
            o[...] = r

    res = pl.pallas_call(body, grid=grid, in_specs=in_specs,
                         out_specs=out_specs, out_shape=out_shape)(*blocked, *full)
    return res


def _encoder_body(ru_ref, emb_ref, w_ref, b_ref, out_ref):
    r = ru_ref[...]
    parts = []
    for c in range(3):
        xc = jnp.zeros(r.shape, jnp.float32)
        for v in range(5):
            xc = xc + jnp.where(r == v, emb_ref[v, c], 0.0)
        parts.append(xc)
    x = jnp.concatenate(parts, axis=-1)
    bn = x.shape[0]
    xr = x.reshape(bn * TP, 3 * K)
    y = jnp.dot(xr, w_ref[...], preferred_element_type=jnp.float32)
    y = jnp.maximum(y + b_ref[...], 0.0)
    out_ref[...] = jnp.max(y.reshape(bn, TP, D), axis=1)


def _encode(reads, emb, conv_w, conv_b):
    ru = jnp.stack([reads[:, k:k + T] for k in range(K)], axis=-1)
    ru = jnp.concatenate([ru, ru[:, :TP - T, :]], axis=1)
    wp = conv_w.transpose(1, 2, 0).reshape(3 * K, D)
    bn = 80
    grid = (pl.cdiv(N, bn),)
    h = pl.pallas_call(
        _encoder_body, grid=grid,
        in_specs=[
            pl.BlockSpec((bn, TP, K), lambda i: (i, 0, 0)),
            pl.BlockSpec(memory_space=pltpu.SMEM),
            pl.BlockSpec((3 * K, D), lambda i: (0, 0)),
            pl.BlockSpec((1, D), lambda i: (0, 0)),
        ],
        out_specs=pl.BlockSpec((bn, D), lambda i: (i, 0)),
        out_shape=jax.ShapeDtypeStruct((N, D), jnp.float32),
    )(ru, emb, wp, conv_b.reshape(1, D))
    return h


def _sigmoid(x):
    return 1.0 / (1.0 + jnp.exp(-x))


def kernel(reads, edge_index, overlap_similarity, overlap_length, emb, conv_w,
           conv_b, edge_w, edge_b, A_w, A_b, B_w, B_b, C_w, C_b, Dm_w, Dm_b,
           Em_w, Em_b, dec_w1, dec_b1, dec_w2, dec_b2):
    E = overlap_similarity.shape[0]
    reads = reads.astype(jnp.int32)
    src = edge_index[0]
    dst = edge_index[1]

    h = _encode(reads, emb, conv_w, conv_b)

    ov = jnp.stack([overlap_similarity, overlap_length], axis=1)

    def edge_enc(o, ew, eb):
        return o[:, 0:1] * ew[0:1, :] + o[:, 1:2] * ew[1:2, :] + eb

    (e,) = _row_call(edge_enc, E, [D], [ov], [edge_w, edge_b.reshape(1, D)], 2048)

    def node_mm(hv, w4, b4):
        return jnp.dot(hv, w4, preferred_element_type=jnp.float32) + b4

    def edge_stage(ev, de, bhs, cw, cb):
        ce = jnp.dot(ev, cw, preferred_element_type=jnp.float32) + cb
        en = ce + de
        sig = _sigmoid(en)
        return (jnp.concatenate([sig, sig * bhs], axis=1),
                ev + jnp.maximum(en, 0.0))

    def h_update(hv, ah, num, den):
        return hv + jnp.maximum(ah + num / (den + 1e-6), 0.0)

    for l in range(4):
        w4 = jnp.concatenate([A_w[l], B_w[l], Dm_w[l], Em_w[l]], axis=1)
        b4 = jnp.concatenate([A_b[l], B_b[l], Dm_b[l], Em_b[l]]).reshape(1, 4 * D)
        (hw,) = _row_call(node_mm, N, [4 * D], [h], [w4, b4], 1024)
        Ah, Bh, Dh, Eh = (hw[:, 0:D], hw[:, D:2 * D], hw[:, 2 * D:3 * D],
                          hw[:, 3 * D:4 * D])
        de = jnp.take(Dh, src, axis=0) + jnp.take(Eh, dst, axis=0)
        bhs = jnp.take(Bh, src, axis=0)
        sgcat, e = _row_call(edge_stage, E, [2 * D, D], [e, de, bhs],
                             [C_w[l], C_b[l].reshape(1, D)], 2048)
        ssum = jax.ops.segment_sum(sgcat, dst, num_segments=N)
        den, num = ssum[:, 0:D], ssum[:, D:2 * D]
        (h,) = _row_call(h_update, N, [D], [h, Ah, num, den], [], 1024)

    hs = jnp.take(h, src, axis=0)
    hd = jnp.take(h, dst, axis=0)

    def dec(a, b, c, w1a, w1b, w1c, b1, w2r, b2):
        z = (jnp.dot(a, w1a, preferred_element_type=jnp.float32)
             + jnp.dot(b, w1b, preferred_element_type=jnp.float32)
             + jnp.dot(c, w1c, preferred_element_type=jnp.float32) + b1)
        z = jnp.maximum(z, 0.0)
        return jnp.sum(z * w2r, axis=1, keepdims=True) + b2

    (p,) = _row_call(dec, E, [1], [hs, hd, e],
                     [dec_w1[0:D], dec_w1[D:2 * D], dec_w1[2 * D:3 * D],
                      dec_b1.reshape(1, D), dec_w2.reshape(1, D),
                      dec_b2.reshape(1, 1)], 1024)
    return p[:, 0]


# speedup vs baseline: 1.1726x; 1.0025x over previous
import functools
import jax
import jax.numpy as jnp
from jax.experimental import pallas as pl
from jax.experimental.pallas import tpu as pltpu

N = 10000
D = 128
L = 128
K = 20
T = L - K + 1
TP = 112


def _row_call(fn, nrows, out_cols, blocked, full, bm):
    grid = (pl.cdiv(nrows, bm),)
    in_specs = [pl.BlockSpec((bm, a.shape[1]), lambda i: (i, 0)) for a in blocked]
    in_specs += [pl.BlockSpec(f.shape, lambda i, _nd=f.ndim: (0,) * _nd) for f in full]
    out_specs = [pl.BlockSpec((bm, c), lambda i: (i, 0)) for c in out_cols]
    out_shape = [jax.ShapeDtypeStruct((nrows, c), jnp.float32) for c in out_cols]

    def body(*refs):
        ins = refs[: len(blocked) + len(full)]
        outs = refs[len(blocked) + len(full):]
        vals = [r[...] for r in ins]
        res = fn(*vals)
        if not isinstance(res, (tuple, list)):
            res = (res,)
        for o, r in zip(outs, res):
            o[...] = r

    res = pl.pallas_call(body, grid=grid, in_specs=in_specs,
                         out_specs=out_specs, out_shape=out_shape)(*blocked, *full)
    return res


def _encoder_body(ru_ref, emb_ref, w_ref, b_ref, out_ref):
    r = ru_ref[...]
    parts = []
    for c in range(3):
        xc = jnp.zeros(r.shape, jnp.float32)
        for v in range(5):
            xc = xc + jnp.where(r == v, emb_ref[v, c], 0.0)
        parts.append(xc)
    x = jnp.concatenate(parts, axis=-1)
    bn = x.shape[0]
    xr = x.reshape(bn * TP, 3 * K)
    y = jnp.dot(xr, w_ref[...], preferred_element_type=jnp.float32)
    y = jnp.maximum(y + b_ref[...], 0.0)
    out_ref[...] = jnp.max(y.reshape(bn, TP, D), axis=1)


def _encode(reads, emb, conv_w, conv_b):
    ru = jnp.stack([reads[:, k:k + T] for k in range(K)], axis=-1)
    ru = jnp.concatenate([ru, ru[:, :TP - T, :]], axis=1)
    wp = conv_w.transpose(1, 2, 0).reshape(3 * K, D)
    bn = 80
    grid = (pl.cdiv(N, bn),)
    h = pl.pallas_call(
        _encoder_body, grid=grid,
        in_specs=[
            pl.BlockSpec((bn, TP, K), lambda i: (i, 0, 0)),
            pl.BlockSpec(memory_space=pltpu.SMEM),
            pl.BlockSpec((3 * K, D), lambda i: (0, 0)),
            pl.BlockSpec((1, D), lambda i: (0, 0)),
        ],
        out_specs=pl.BlockSpec((bn, D), lambda i: (i, 0)),
        out_shape=jax.ShapeDtypeStruct((N, D), jnp.float32),
    )(ru, emb, wp, conv_b.reshape(1, D))
    return h


def _sigmoid(x):
    return 1.0 / (1.0 + jnp.exp(-x))


def kernel(reads, edge_index, overlap_similarity, overlap_length, emb, conv_w,
           conv_b, edge_w, edge_b, A_w, A_b, B_w, B_b, C_w, C_b, Dm_w, Dm_b,
           Em_w, Em_b, dec_w1, dec_b1, dec_w2, dec_b2):
    E = overlap_similarity.shape[0]
    reads = reads.astype(jnp.int32)
    src = edge_index[0]
    dst = edge_index[1]

    h = _encode(reads, emb, conv_w, conv_b)

    ov = jnp.stack([overlap_similarity, overlap_length], axis=1)

    def edge_enc(o, ew, eb):
        return o[:, 0:1] * ew[0:1, :] + o[:, 1:2] * ew[1:2, :] + eb

    (e,) = _row_call(edge_enc, E, [D], [ov], [edge_w, edge_b.reshape(1, D)], 2048)

    def node_mm(hv, w4, b4):
        return jnp.dot(hv, w4, preferred_element_type=jnp.float32) + b4

    def edge_stage(ev, de, bhs, cw, cb):
        ce = jnp.dot(ev, cw, preferred_element_type=jnp.float32) + cb
        en = ce + de
        sig = _sigmoid(en)
        return (jnp.concatenate([sig, sig * bhs], axis=1),
                ev + jnp.maximum(en, 0.0))

    def h_update(hv, ah, num, den):
        return hv + jnp.maximum(ah + num / (den + 1e-6), 0.0)

    for l in range(4):
        w4 = jnp.concatenate([A_w[l], B_w[l], Dm_w[l], Em_w[l]], axis=1)
        b4 = jnp.concatenate([A_b[l], B_b[l], Dm_b[l], Em_b[l]]).reshape(1, 4 * D)
        (hw,) = _row_call(node_mm, N, [4 * D], [h], [w4, b4], 1024)
        Ah, Bh, Dh, Eh = (hw[:, 0:D], hw[:, D:2 * D], hw[:, 2 * D:3 * D],
                          hw[:, 3 * D:4 * D])
        de = jnp.take(Dh, src, axis=0) + jnp.take(Eh, dst, axis=0)
        bhs = jnp.take(Bh, src, axis=0)
        sgcat, e = _row_call(edge_stage, E, [2 * D, D], [e, de, bhs],
                             [C_w[l], C_b[l].reshape(1, D)], 2048)
        ssum = jax.ops.segment_sum(sgcat, dst, num_segments=N)
        den, num = ssum[:, 0:D], ssum[:, D:2 * D]
        (h,) = _row_call(h_update, N, [D], [h, Ah, num, den], [], 1024)

    w12 = jnp.concatenate([dec_w1[0:D], dec_w1[D:2 * D]], axis=1)
    (hp,) = _row_call(node_mm, N, [2 * D], [h],
                      [w12, jnp.zeros((1, 2 * D), jnp.float32)], 1024)
    g = jnp.take(hp[:, 0:D], src, axis=0) + jnp.take(hp[:, D:2 * D], dst, axis=0)

    def dec(ev, gv, w1c, b1, w2r, b2):
        z = gv + jnp.dot(ev, w1c, preferred_element_type=jnp.float32) + b1
        z = jnp.maximum(z, 0.0)
        return jnp.sum(z * w2r, axis=1, keepdims=True) + b2

    (p,) = _row_call(dec, E, [1], [e, g],
                     [dec_w1[2 * D:3 * D], dec_b1.reshape(1, D),
                      dec_w2.reshape(1, D), dec_b2.reshape(1, 1)], 2048)
    return p[:, 0]
